# Initial kernel scaffold; baseline (speedup 1.0000x reference)
import functools

import jax
import jax.numpy as jnp
import numpy as np
from jax import lax
from jax.experimental import pallas as pl
from jax.experimental.pallas import tpu as pltpu
from jax.experimental.pallas import tpu_sc as plsc

MAX_SLOT_K = 20
N_SLOTS = 18
SLOT0 = 1
EMB_D = 16
LANES = 16
CHUNK = 128
PERIOD = 144
ROWS_PER_PERIOD = PERIOD // N_SLOTS


def _pattern_tables():
    t = np.arange(PERIOD, dtype=np.int32)
    r, j = t // N_SLOTS, t % N_SLOTS
    pos = (r * MAX_SLOT_K + j + SLOT0).astype(np.int32)
    return pos, j.astype(np.int32)


def _body(num_emb, n_chunks, in_hbm, pos_hbm, joff_hbm, tab_hbm, out_hbm,
          raw_v, pos_v, joff_v, idx_v, rows_v, sem):
    nc = 2
    wid = lax.axis_index("s") * nc + lax.axis_index("c")
    rows_per_w = n_chunks * CHUNK
    base_row = wid * rows_per_w

    pltpu.sync_copy(pos_hbm, pos_v)
    pltpu.sync_copy(joff_hbm, joff_v)

    def chunk_body(c, carry):
        row0 = base_row + c * CHUNK
        pltpu.sync_copy(in_hbm.at[pl.ds(row0 * MAX_SLOT_K, CHUNK * MAX_SLOT_K)],
                        raw_v)
        for p in range(CHUNK // ROWS_PER_PERIOD):
            pbase = p * ROWS_PER_PERIOD * MAX_SLOT_K
            for g in range(PERIOD // LANES):
                pos = pos_v[pl.ds(g * LANES, LANES)] + pbase
                vals = plsc.load_gather(raw_v, [pos])
                idx = vals + joff_v[pl.ds(g * LANES, LANES)] * num_emb
                idx_v[pl.ds(p * PERIOD + g * LANES, LANES)] = idx
        copies = []
        for k in range(N_SLOTS):
            copies.append(pltpu.async_copy(
                tab_hbm.at[idx_v.at[pl.ds(k * CHUNK, CHUNK)]],
                rows_v.at[pl.ds(k * CHUNK, CHUNK)],
                sem))
        for cp in copies:
            cp.wait()
        pltpu.sync_copy(rows_v, out_hbm.at[pl.ds(row0 * N_SLOTS,
                                                 CHUNK * N_SLOTS)])
        return carry

    lax.fori_loop(0, n_chunks, chunk_body, 0)


def kernel(sequence_bucket_inputs, tables):
    b, l, k = sequence_bucket_inputs.shape
    n_tab, num_emb, d = tables.shape
    assert k == MAX_SLOT_K and n_tab == N_SLOTS and d == EMB_D
    n_rows = b * l
    n_workers = 32
    assert n_rows % (n_workers * CHUNK) == 0
    n_chunks = n_rows // (n_workers * CHUNK)

    flat_in = sequence_bucket_inputs.astype(jnp.int32).reshape(-1)
    flat_tab = tables.reshape(n_tab * num_emb, d)
    pos_pat, joff_pat = _pattern_tables()

    mesh = plsc.VectorSubcoreMesh(core_axis_name="c", subcore_axis_name="s")
    run = pl.kernel(
        functools.partial(_body, num_emb, n_chunks),
        out_type=jax.ShapeDtypeStruct((n_rows * N_SLOTS, EMB_D), jnp.float32),
        mesh=mesh,
        scratch_types=[
            pltpu.VMEM((CHUNK * MAX_SLOT_K,), jnp.int32),
            pltpu.VMEM((PERIOD,), jnp.int32),
            pltpu.VMEM((PERIOD,), jnp.int32),
            pltpu.VMEM((CHUNK * N_SLOTS,), jnp.int32),
            pltpu.VMEM((CHUNK * N_SLOTS, EMB_D), jnp.float32),
            pltpu.SemaphoreType.DMA,
        ],
    )
    out = run(flat_in, jnp.asarray(pos_pat), jnp.asarray(joff_pat), flat_tab)
    return out.reshape(b, l, N_SLOTS * EMB_D)


# speedup vs baseline: 2.4707x; 2.4707x over previous
import functools

import jax
import jax.numpy as jnp
import numpy as np
from jax import lax
from jax.experimental import pallas as pl
from jax.experimental.pallas import tpu as pltpu
from jax.experimental.pallas import tpu_sc as plsc

MAX_SLOT_K = 20
N_SLOTS = 18
SLOT0 = 1
EMB_D = 16
LANES = 16
CHUNK = 128
PERIOD = 144
ROWS_PER_PERIOD = PERIOD // N_SLOTS


def _pattern_tables():
    t = np.arange(PERIOD, dtype=np.int32)
    r, j = t // N_SLOTS, t % N_SLOTS
    pos = (r * MAX_SLOT_K + j + SLOT0).astype(np.int32)
    return pos, j.astype(np.int32)


def _body(num_emb, n_chunks, in_hbm, pos_hbm, joff_hbm, tab_hbm, out_hbm,
          raw_v, pos_v, joff_v, idx_v, rows_v, sem):
    nc = 2
    wid = lax.axis_index("s") * nc + lax.axis_index("c")
    rows_per_w = n_chunks * CHUNK
    base_row = wid * rows_per_w

    pltpu.sync_copy(pos_hbm, pos_v)
    pltpu.sync_copy(joff_hbm, joff_v)

    def chunk_body(c, carry):
        row0 = base_row + c * CHUNK
        pltpu.sync_copy(in_hbm.at[pl.ds(row0 * MAX_SLOT_K, CHUNK * MAX_SLOT_K)],
                        raw_v)
        for p in range(CHUNK // ROWS_PER_PERIOD):
            pbase = p * ROWS_PER_PERIOD * MAX_SLOT_K
            for g in range(PERIOD // LANES):
                pos = pos_v[pl.ds(g * LANES, LANES)] + pbase
                vals = plsc.load_gather(raw_v, [pos])
                idx = vals + joff_v[pl.ds(g * LANES, LANES)] * num_emb
                grp = 9 * p + g
                idx_v[grp // 8, pl.ds((grp % 8) * LANES, LANES)] = idx
        copies = []
        for k in range(N_SLOTS):
            copies.append(pltpu.async_copy(
                tab_hbm.at[idx_v.at[k]],
                rows_v.at[pl.ds(k * CHUNK, CHUNK)],
                sem))
        for cp in copies:
            cp.wait()
        pltpu.sync_copy(rows_v, out_hbm.at[pl.ds(row0 * N_SLOTS,
                                                 CHUNK * N_SLOTS)])
        return carry

    lax.fori_loop(0, n_chunks, chunk_body, 0)


def kernel(sequence_bucket_inputs, tables):
    b, l, k = sequence_bucket_inputs.shape
    n_tab, num_emb, d = tables.shape
    assert k == MAX_SLOT_K and n_tab == N_SLOTS and d == EMB_D
    n_rows = b * l
    n_workers = 32
    assert n_rows % (n_workers * CHUNK) == 0
    n_chunks = n_rows // (n_workers * CHUNK)

    flat_in = sequence_bucket_inputs.astype(jnp.int32).reshape(-1)
    flat_tab = tables.reshape(n_tab * num_emb, d)
    pos_pat, joff_pat = _pattern_tables()

    mesh = plsc.VectorSubcoreMesh(core_axis_name="c", subcore_axis_name="s",
                                  num_cores=2, num_subcores=16)
    run = pl.kernel(
        functools.partial(_body, num_emb, n_chunks),
        out_type=jax.ShapeDtypeStruct((n_rows * N_SLOTS, EMB_D), jnp.float32),
        mesh=mesh,
        compiler_params=pltpu.CompilerParams(needs_layout_passes=False,
                                             use_tc_tiling_on_sc=False),
        scratch_types=[
            pltpu.VMEM((CHUNK * MAX_SLOT_K,), jnp.int32),
            pltpu.VMEM((PERIOD,), jnp.int32),
            pltpu.VMEM((PERIOD,), jnp.int32),
            pltpu.VMEM((N_SLOTS, CHUNK), jnp.int32),
            pltpu.VMEM((CHUNK * N_SLOTS, EMB_D), jnp.float32),
            pltpu.SemaphoreType.DMA,
        ],
    )
    out = run(flat_in, jnp.asarray(pos_pat), jnp.asarray(joff_pat), flat_tab)
    return out.reshape(b, l, N_SLOTS * EMB_D)


# speedup vs baseline: 2.4941x; 1.0095x over previous
import functools

import jax
import jax.numpy as jnp
import numpy as np
from jax import lax
from jax.experimental import pallas as pl
from jax.experimental.pallas import tpu as pltpu
from jax.experimental.pallas import tpu_sc as plsc

MAX_SLOT_K = 20
N_SLOTS = 18
SLOT0 = 1
EMB_D = 16
LANES = 16
CHUNK = 128
PERIOD = 144
ROWS_PER_PERIOD = PERIOD // N_SLOTS
SLAB = CHUNK * MAX_SLOT_K


def _pattern_tables(num_emb):
    t = np.arange(PERIOD, dtype=np.int32)
    r, j = t // N_SLOTS, t % N_SLOTS
    pos = (r * MAX_SLOT_K + j + SLOT0).astype(np.int32)
    return pos, (j * num_emb).astype(np.int32)


def _body(n_chunks, in_hbm, pos_hbm, joff_hbm, tab_hbm, out_hbm,
          raw_v, pos_v, joff_v, idx_v, rows_v, ssem0, ssem1, gsem,
          wsem0, wsem1):
    nc = 2
    wid = lax.axis_index("s") * nc + lax.axis_index("c")
    base_row = wid * n_chunks * CHUNK
    n_pairs = n_chunks // 2

    pltpu.sync_copy(pos_hbm, pos_v)
    pltpu.sync_copy(joff_hbm, joff_v)
    pats = [(pos_v[pl.ds(g * LANES, LANES)], joff_v[pl.ds(g * LANES, LANES)])
            for g in range(PERIOD // LANES)]

    def start_slab(c, b, sem):
        return pltpu.async_copy(
            in_hbm.at[pl.ds((base_row + c * CHUNK) * MAX_SLOT_K, SLAB)],
            raw_v.at[b], sem)

    def wait_slab(b, sem):
        pltpu.make_async_copy(in_hbm.at[pl.ds(0, SLAB)], raw_v.at[b],
                              sem).wait()

    def compute_idx(b):
        for p in range(CHUNK // ROWS_PER_PERIOD):
            pbase = p * ROWS_PER_PERIOD * MAX_SLOT_K
            for g in range(PERIOD // LANES):
                pos_g, joff_g = pats[g]
                vals = plsc.load_gather(raw_v.at[b], [pos_g + pbase])
                grp = 9 * p + g
                idx_v[b, grp // 8, pl.ds((grp % 8) * LANES, LANES)] = (
                    vals + joff_g)

    def step(i, c, b, ssem_nxt, wsem, b_nxt, prefetch=True):
        cid = base_row // CHUNK + c
        if prefetch:
            start_slab(c + 1, b_nxt, ssem_nxt)
        compute_idx(b)

        def _wait_prev_writeback():
            pltpu.make_async_copy(rows_v.at[b], out_hbm.at[cid], wsem).wait()

        if isinstance(i, int):
            if i > 0:
                _wait_prev_writeback()
        else:
            pl.when(i > 0)(_wait_prev_writeback)

        copies = [pltpu.async_copy(tab_hbm.at[idx_v.at[b, k]],
                                   rows_v.at[b, k], gsem)
                  for k in range(N_SLOTS)]
        for cp in copies:
            cp.wait()
        pltpu.async_copy(rows_v.at[b], out_hbm.at[cid], wsem)

    start_slab(0, 0, ssem0)

    def pair_body(i, carry):
        c0 = i * 2
        wait_slab(0, ssem0)
        step(i, c0, 0, ssem1, wsem0, 1)
        wait_slab(1, ssem1)
        step(i, c0 + 1, 1, ssem0, wsem1, 0)
        return carry

    if n_pairs > 1:
        lax.fori_loop(0, n_pairs - 1, pair_body, 0)
    ilast = n_pairs - 1
    wait_slab(0, ssem0)
    step(ilast, 2 * ilast, 0, ssem1, wsem0, 1)
    wait_slab(1, ssem1)
    step(ilast, 2 * ilast + 1, 1, ssem0, wsem1, 0, prefetch=False)
    pltpu.make_async_copy(rows_v.at[0], out_hbm.at[0], wsem0).wait()
    pltpu.make_async_copy(rows_v.at[1], out_hbm.at[0], wsem1).wait()


def kernel(sequence_bucket_inputs, tables):
    b, l, k = sequence_bucket_inputs.shape
    n_tab, num_emb, d = tables.shape
    assert k == MAX_SLOT_K and n_tab == N_SLOTS and d == EMB_D
    n_rows = b * l
    n_workers = 32
    assert n_rows % (n_workers * CHUNK * 2) == 0
    n_chunks = n_rows // (n_workers * CHUNK)

    flat_in = sequence_bucket_inputs.astype(jnp.int32).reshape(-1)
    flat_tab = tables.reshape(n_tab * num_emb, d)
    pos_pat, joff_pat = _pattern_tables(num_emb)

    mesh = plsc.VectorSubcoreMesh(core_axis_name="c", subcore_axis_name="s",
                                  num_cores=2, num_subcores=16)
    run = pl.kernel(
        functools.partial(_body, n_chunks),
        out_type=jax.ShapeDtypeStruct((n_rows // CHUNK, N_SLOTS, CHUNK, EMB_D),
                                      jnp.float32),
        mesh=mesh,
        compiler_params=pltpu.CompilerParams(needs_layout_passes=False,
                                             use_tc_tiling_on_sc=False),
        scratch_types=[
            pltpu.VMEM((2, SLAB), jnp.int32),
            pltpu.VMEM((PERIOD,), jnp.int32),
            pltpu.VMEM((PERIOD,), jnp.int32),
            pltpu.VMEM((2, N_SLOTS, CHUNK), jnp.int32),
            pltpu.VMEM((2, N_SLOTS, CHUNK, EMB_D), jnp.float32),
            pltpu.SemaphoreType.DMA,
            pltpu.SemaphoreType.DMA,
            pltpu.SemaphoreType.DMA,
            pltpu.SemaphoreType.DMA,
            pltpu.SemaphoreType.DMA,
        ],
    )
    out = run(flat_in, jnp.asarray(pos_pat), jnp.asarray(joff_pat), flat_tab)
    return out.reshape(b, l, N_SLOTS * EMB_D)


# speedup vs baseline: 4.6159x; 1.8507x over previous
import functools

import jax
import jax.numpy as jnp
import numpy as np
from jax import lax
from jax.experimental import pallas as pl
from jax.experimental.pallas import tpu as pltpu
from jax.experimental.pallas import tpu_sc as plsc

MAX_SLOT_K = 20
N_SLOTS = 18
SLOT0 = 1
EMB_D = 16
LANES = 16
CHUNK = 128
PERIOD = 144
ROWS_PER_PERIOD = PERIOD // N_SLOTS
SLAB = CHUNK * MAX_SLOT_K


def _pattern_tables(num_emb):
    t = np.arange(PERIOD, dtype=np.int32)
    r, j = t // N_SLOTS, t % N_SLOTS
    pos = (r * MAX_SLOT_K + j + SLOT0).astype(np.int32)
    return pos, (j * num_emb).astype(np.int32)


def _body(n_chunks, tab_rows, in_hbm, pos_hbm, joff_hbm, tab_hbm, out_hbm,
          raw_v, pos_v, joff_v, idx_v, rows_v, ssem0, ssem1, gsem,
          wsem0, wsem1):
    nc = 2
    wid = lax.axis_index("s") * nc + lax.axis_index("c")
    base_row = wid * n_chunks * CHUNK
    n_pairs = n_chunks // 2

    pltpu.sync_copy(pos_hbm, pos_v)
    pltpu.sync_copy(joff_hbm, joff_v)
    pats = [(pos_v[pl.ds(g * LANES, LANES)], joff_v[pl.ds(g * LANES, LANES)])
            for g in range(PERIOD // LANES)]

    def start_slab(c, b, sem):
        return pltpu.async_copy(
            in_hbm.at[pl.ds((base_row + c * CHUNK) * MAX_SLOT_K, SLAB)],
            raw_v.at[b], sem)

    def wait_slab(b, sem):
        pltpu.make_async_copy(in_hbm.at[pl.ds(0, SLAB)], raw_v.at[b],
                              sem).wait()

    def compute_idx(b):
        for p in range(CHUNK // ROWS_PER_PERIOD):
            pbase = p * ROWS_PER_PERIOD * MAX_SLOT_K
            for g in range(PERIOD // LANES):
                pos_g, joff_g = pats[g]
                vals = plsc.load_gather(raw_v.at[b], [pos_g + pbase])
                grp = 9 * p + g
                idx_v[b, grp // 8, pl.ds((grp % 8) * LANES, LANES)] = (
                    vals + joff_g)

    def step(i, c, b, ssem_nxt, wsem, b_nxt, prefetch=True):
        cid = base_row // CHUNK + c
        if prefetch:
            start_slab(c + 1, b_nxt, ssem_nxt)
        compute_idx(b)

        def _wait_prev_writeback():
            pltpu.make_async_copy(rows_v.at[b], out_hbm.at[cid], wsem).wait()

        if isinstance(i, int):
            if i > 0:
                _wait_prev_writeback()
        else:
            pl.when(i > 0)(_wait_prev_writeback)

        copies = [pltpu.async_copy(tab_hbm.at[idx_v.at[b, k]],
                                   rows_v.at[b, k], gsem)
                  for k in range(N_SLOTS)]
        for cp in copies:
            cp.wait()
        pltpu.async_copy(rows_v.at[b], out_hbm.at[cid], wsem)

    start_slab(0, 0, ssem0)

    def pair_body(i, carry):
        c0 = i * 2
        wait_slab(0, ssem0)
        step(i, c0, 0, ssem1, wsem0, 1)
        wait_slab(1, ssem1)
        step(i, c0 + 1, 1, ssem0, wsem1, 0)
        return carry

    if n_pairs > 1:
        lax.fori_loop(0, n_pairs - 1, pair_body, 0)
    ilast = n_pairs - 1
    wait_slab(0, ssem0)
    step(ilast, 2 * ilast, 0, ssem1, wsem0, 1)
    wait_slab(1, ssem1)
    step(ilast, 2 * ilast + 1, 1, ssem0, wsem1, 0, prefetch=False)
    pltpu.make_async_copy(rows_v.at[0], out_hbm.at[0], wsem0).wait()
    pltpu.make_async_copy(rows_v.at[1], out_hbm.at[0], wsem1).wait()


def kernel(sequence_bucket_inputs, tables):
    b, l, k = sequence_bucket_inputs.shape
    n_tab, num_emb, d = tables.shape
    assert k == MAX_SLOT_K and n_tab == N_SLOTS and d == EMB_D
    n_rows = b * l
    n_workers = 32
    assert n_rows % (n_workers * CHUNK * 2) == 0
    n_chunks = n_rows // (n_workers * CHUNK)

    flat_in = sequence_bucket_inputs.astype(jnp.int32).reshape(-1)
    tab_1d = lax.optimization_barrier(tables.reshape(-1)).reshape(
        n_tab * num_emb, d)
    pos_pat, joff_pat = _pattern_tables(num_emb)

    mesh = plsc.VectorSubcoreMesh(core_axis_name="c", subcore_axis_name="s",
                                  num_cores=2, num_subcores=16)
    run = pl.kernel(
        functools.partial(_body, n_chunks, n_tab * num_emb),
        out_type=jax.ShapeDtypeStruct((n_rows // CHUNK, N_SLOTS, CHUNK, EMB_D),
                                      jnp.float32),
        mesh=mesh,
        compiler_params=pltpu.CompilerParams(needs_layout_passes=False,
                                             use_tc_tiling_on_sc=False),
        scratch_types=[
            pltpu.VMEM((2, SLAB), jnp.int32),
            pltpu.VMEM((PERIOD,), jnp.int32),
            pltpu.VMEM((PERIOD,), jnp.int32),
            pltpu.VMEM((2, N_SLOTS, CHUNK), jnp.int32),
            pltpu.VMEM((2, N_SLOTS, CHUNK, EMB_D), jnp.float32),
            pltpu.SemaphoreType.DMA,
            pltpu.SemaphoreType.DMA,
            pltpu.SemaphoreType.DMA,
            pltpu.SemaphoreType.DMA,
            pltpu.SemaphoreType.DMA,
        ],
    )
    out = run(flat_in, jnp.asarray(pos_pat), jnp.asarray(joff_pat), tab_1d)
    return out.reshape(b, l, N_SLOTS * EMB_D)
